# baseline (speedup 1.0000x reference)
import functools

import jax
import jax.numpy as jnp
from jax import lax
from jax.experimental import pallas as pl
from jax.experimental.pallas import tpu as pltpu
from jax.experimental.pallas import tpu_sc as plsc

N = 4096
U = 512
V = U + 1
BOS = 1
EOS = 2

NC = 2
NS = 16
NW = NC * NS
IB = N // NW
NJ = U // 128

_mesh = plsc.VectorSubcoreMesh(core_axis_name="c", subcore_axis_name="s")


@functools.partial(
    pl.kernel,
    out_type=[
        jax.ShapeDtypeStruct((V, N), jnp.int32),
        jax.ShapeDtypeStruct((V, N), jnp.int32),
    ],
    mesh=_mesh,
    scratch_types=[
        pltpu.VMEM((IB, 128), jnp.int32),
        pltpu.VMEM((129, IB), jnp.int32),
        pltpu.VMEM((1, IB), jnp.int32),
        pltpu.VMEM((1, IB), jnp.int32),
        pltpu.VMEM((IB,), jnp.int32),
    ],
    compiler_params=pltpu.CompilerParams(needs_layout_passes=False),
)
def _seq_extract(y_hbm, ly_hbm, zp_hbm, zt_hbm, ybuf, wbuf, carry, zrow, lybuf):
    wid = lax.axis_index("s") * NC + lax.axis_index("c")
    i0 = wid * IB
    iota = lax.iota(jnp.int32, 16)
    eosv = jnp.full((16,), EOS, jnp.int32)
    zeros16 = jnp.zeros((16,), jnp.int32)

    pltpu.sync_copy(ly_hbm.at[pl.ds(i0, IB)], lybuf)
    for u in range(IB // 16):
        zrow[0, pl.ds(u * 16, 16)] = zeros16

    for jt in range(NJ):
        pltpu.sync_copy(y_hbm.at[pl.ds(i0, IB), pl.ds(jt * 128, 128)], ybuf)

        for u in range(IB // 16):
            if jt == 0:
                wbuf[0, pl.ds(u * 16, 16)] = jnp.full((16,), BOS, jnp.int32)
            else:
                wbuf[0, pl.ds(u * 16, 16)] = carry[0, pl.ds(u * 16, 16)]

        @plsc.parallel_loop(0, IB, unroll=2)
        def _transpose(r):
            rv = jnp.full((16,), r, jnp.int32)
            vs = [ybuf[r, pl.ds(u * 16, 16)] for u in range(8)]
            for u in range(8):
                plsc.store_scatter(wbuf, [u * 16 + iota + 1, rv], vs[u])

        for u in range(IB // 16):
            carry[0, pl.ds(u * 16, 16)] = wbuf[128, pl.ds(u * 16, 16)]

        pltpu.sync_copy(wbuf.at[pl.ds(0, 128), :],
                        zp_hbm.at[pl.ds(jt * 128, 128), pl.ds(i0, IB)])

        lo = jt * 128
        for g in range(IB // 16):
            lyv = lybuf[pl.ds(g * 16, 16)]
            m = (lyv >= lo) & (lyv < lo + 128)
            plsc.store_scatter(wbuf, [lyv - lo + 1, g * 16 + iota], eosv, mask=m)

        pltpu.sync_copy(wbuf.at[pl.ds(1, 128), :],
                        zt_hbm.at[pl.ds(jt * 128, 128), pl.ds(i0, IB)])

    pltpu.sync_copy(carry, zp_hbm.at[pl.ds(U, 1), pl.ds(i0, IB)])
    pltpu.sync_copy(zrow, zt_hbm.at[pl.ds(U, 1), pl.ds(i0, IB)])


def kernel(y, ly):
    zp, zt = _seq_extract(y, ly)
    return zp.T, zt.T
